# baseline (speedup 1.0000x reference)
import functools

import jax
import jax.numpy as jnp
from jax.experimental import pallas as pl

_S_NUM = 2048
_T_NUM = 512
_K_LOW = 943717
_MAX_FINITE_BITS = 0x7F7FFFFF


def _body(s_ref, t_ref, w1_ref, b1_ref, w2_ref, b2_ref, w_ref, bias_ref,
          task_ref, out_ref):
    f32 = jnp.float32
    S = s_ref[...]
    T = t_ref[...]

    dot = functools.partial(jax.lax.dot_general,
                            preferred_element_type=jnp.float32)

    scores = jnp.maximum(
        dot(S, T, (((1,), (1,)), ((), ()))), 0.0)

    ka = jnp.int32(_K_LOW + 1)
    kb = jnp.int32(_K_LOW + 2)
    maxf = jnp.int32(_MAX_FINITE_BITS)

    def tri_step(_, carry):
        lo_a, lo_b, w = carry
        q1 = w // 3
        q2 = (2 * w) // 3

        def advance(lo, K):
            b1 = jnp.minimum(lo + q1 - 1, maxf)
            b2 = jnp.minimum(lo + q2 - 1, maxf)
            t1 = jax.lax.bitcast_convert_type(b1, f32)
            t2 = jax.lax.bitcast_convert_type(b2, f32)
            c1 = jnp.count_nonzero(scores <= t1)
            c2 = jnp.count_nonzero(scores <= t2)
            lo = jnp.where(c1 < K, jnp.where(c2 < K, lo + q2, lo + q1), lo)
            return lo

        w_next = jnp.maximum(jnp.maximum(q1, q2 - q1), w - q2)
        return advance(lo_a, ka), advance(lo_b, kb), w_next

    lo0 = jnp.int32(0)
    w0 = jnp.int32(0x7FFFFFFF)
    lo_a, lo_b, w_fin = jax.lax.fori_loop(
        0, 21, tri_step, (lo0, lo0, w0))

    def bin_step(carry):
        lo_a, lo_b, w = carry
        h = w // 2

        def advance(lo, K):
            t = jax.lax.bitcast_convert_type(
                jnp.minimum(lo + h - 1, maxf), f32)
            c = jnp.count_nonzero(scores <= t)
            return jnp.where(c < K, lo + h, lo)

        return advance(lo_a, ka), advance(lo_b, kb), w - h

    carry = (lo_a, lo_b, w_fin)
    carry = bin_step(carry)
    carry = bin_step(carry)
    vk_bits, vk1_bits, _ = carry

    vk = jax.lax.bitcast_convert_type(vk_bits, f32)
    vk1 = jax.lax.bitcast_convert_type(vk1_bits, f32)
    md = vk + (vk1 - vk) * f32(0.5)

    mask = (scores > md).astype(f32)
    ones_t = jnp.ones((_T_NUM, 1), f32)
    ones_s = jnp.ones((_S_NUM, 1), f32)
    deg_s = dot(mask, ones_t, (((1,), (0,)), ((), ()))) + 1.0
    deg_t = dot(mask, ones_s, (((0,), (0,)), ((), ()))) + 1.0
    dinv_s = jax.lax.rsqrt(deg_s)
    dinv_t = jax.lax.rsqrt(deg_t)

    W1 = w1_ref[...]
    b1 = b1_ref[...]
    W2 = w2_ref[...]
    b2 = b2_ref[...]

    def agg(hs, ht):
        ms = dot(mask, dinv_t * ht, (((1,), (0,)), ((), ())))
        mt = dot(mask, dinv_s * hs, (((0,), (0,)), ((), ())))
        out_s = dinv_s * (dinv_s * hs + ms)
        out_t = dinv_t * (dinv_t * ht + mt)
        return out_s, out_t

    hs1 = dot(S, W1, (((1,), (0,)), ((), ())))
    ht1 = dot(T, W1, (((1,), (0,)), ((), ())))
    as1, at1 = agg(hs1, ht1)
    h1s = jnp.maximum(as1 + b1, 0.0)
    h1t = jnp.maximum(at1 + b1, 0.0)

    hs2 = dot(h1s, W2, (((1,), (0,)), ((), ())))
    ht2 = dot(h1t, W2, (((1,), (0,)), ((), ())))
    emb_s, emb_t = agg(hs2, ht2)
    emb_s = emb_s + b2
    emb_t = emb_t + b2

    tgt = jnp.sum(emb_t, axis=0, keepdims=True) * f32(1.0 / _T_NUM)
    wv = (w_ref[...] * tgt.T)
    soutar = dot(emb_s, wv, (((1,), (0,)), ((), ()))) + bias_ref[...]
    out = 0.5 * jax.nn.sigmoid(soutar) + 0.5 * jax.nn.sigmoid(task_ref[...])
    out_ref[...] = out


@jax.jit
def kernel(source_stack, target_stack, W1, b1, W2, b2, w, b, task_vec):
    out = pl.pallas_call(
        _body,
        out_shape=jax.ShapeDtypeStruct((_S_NUM, 1), jnp.float32),
    )(source_stack, target_stack, W1, b1.reshape(1, -1), W2,
      b2.reshape(1, -1), w, b.reshape(1, 1), task_vec)
    return out


# speedup vs baseline: 1.0918x; 1.0918x over previous
import functools

import jax
import jax.numpy as jnp
from jax.experimental import pallas as pl

_S_NUM = 2048
_T_NUM = 512
_K_LOW = 943717
_MAX_FINITE_BITS = 0x7F7FFFFF


def _body(s_ref, t_ref, w1_ref, b1_ref, w2_ref, b2_ref, w_ref, bias_ref,
          task_ref, out_ref):
    f32 = jnp.float32
    S = s_ref[...]
    T = t_ref[...]

    dot = functools.partial(jax.lax.dot_general,
                            preferred_element_type=jnp.float32)

    scores = jnp.maximum(
        dot(S, T, (((1,), (1,)), ((), ()))), 0.0)

    ka = jnp.int32(_K_LOW + 1)
    kb = jnp.int32(_K_LOW + 2)
    maxf = jnp.int32(_MAX_FINITE_BITS)

    probe_sched = []
    w = 0x7FFFFFFF
    while w > 2:
        q1, q2 = w // 3, (2 * w) // 3
        probe_sched.append((q1, q2))
        w = max(q1, q2 - q1, w - q2)
    probe_sched.append((w // 2, w // 2))

    def advance(lo, K, q1, q2):
        b1 = lo + jnp.minimum(jnp.int32(q1 - 1), maxf - lo)
        b2 = lo + jnp.minimum(jnp.int32(q2 - 1), maxf - lo)
        t1 = jax.lax.bitcast_convert_type(b1, f32)
        t2 = jax.lax.bitcast_convert_type(b2, f32)
        c1 = jnp.count_nonzero(scores <= t1)
        c2 = jnp.count_nonzero(scores <= t2)
        return jnp.where(c1 < K,
                         jnp.where(c2 < K, lo + jnp.int32(q2),
                                   lo + jnp.int32(q1)),
                         lo)

    lo_a = jnp.int32(0)
    lo_b = jnp.int32(0)
    for q1, q2 in probe_sched:
        lo_a = advance(lo_a, ka, q1, q2)
        lo_b = advance(lo_b, kb, q1, q2)
    vk_bits = lo_a
    vk1_bits = lo_b

    vk = jax.lax.bitcast_convert_type(vk_bits, f32)
    vk1 = jax.lax.bitcast_convert_type(vk1_bits, f32)
    md = vk + (vk1 - vk) * f32(0.5)

    mask = (scores > md).astype(f32)
    ones_t = jnp.ones((_T_NUM, 1), f32)
    ones_s = jnp.ones((_S_NUM, 1), f32)
    deg_s = dot(mask, ones_t, (((1,), (0,)), ((), ()))) + 1.0
    deg_t = dot(mask, ones_s, (((0,), (0,)), ((), ()))) + 1.0
    dinv_s = jax.lax.rsqrt(deg_s)
    dinv_t = jax.lax.rsqrt(deg_t)

    W1 = w1_ref[...]
    b1 = b1_ref[...]
    W2 = w2_ref[...]
    b2 = b2_ref[...]

    def agg(hs, ht):
        ms = dot(mask, dinv_t * ht, (((1,), (0,)), ((), ())))
        mt = dot(mask, dinv_s * hs, (((0,), (0,)), ((), ())))
        out_s = dinv_s * (dinv_s * hs + ms)
        out_t = dinv_t * (dinv_t * ht + mt)
        return out_s, out_t

    hs1 = dot(S, W1, (((1,), (0,)), ((), ())))
    ht1 = dot(T, W1, (((1,), (0,)), ((), ())))
    as1, at1 = agg(hs1, ht1)
    h1s = jnp.maximum(as1 + b1, 0.0)
    h1t = jnp.maximum(at1 + b1, 0.0)

    hs2 = dot(h1s, W2, (((1,), (0,)), ((), ())))
    ht2 = dot(h1t, W2, (((1,), (0,)), ((), ())))
    emb_s, emb_t = agg(hs2, ht2)
    emb_s = emb_s + b2
    emb_t = emb_t + b2

    tgt = jnp.sum(emb_t, axis=0, keepdims=True) * f32(1.0 / _T_NUM)
    wv = (w_ref[...] * tgt.T)
    soutar = dot(emb_s, wv, (((1,), (0,)), ((), ()))) + bias_ref[...]
    out = 0.5 * jax.nn.sigmoid(soutar) + 0.5 * jax.nn.sigmoid(task_ref[...])
    out_ref[...] = out


@jax.jit
def kernel(source_stack, target_stack, W1, b1, W2, b2, w, b, task_vec):
    out = pl.pallas_call(
        _body,
        out_shape=jax.ShapeDtypeStruct((_S_NUM, 1), jnp.float32),
    )(source_stack, target_stack, W1, b1.reshape(1, -1), W2,
      b2.reshape(1, -1), w, b.reshape(1, 1), task_vec)
    return out


# speedup vs baseline: 1.2623x; 1.1561x over previous
import functools

import jax
import jax.numpy as jnp
from jax.experimental import pallas as pl

_S_NUM = 2048
_T_NUM = 512
_K_LOW = 943717
_MAX_FINITE_BITS = 0x7F7FFFFF


def _body(s_ref, t_ref, w1_ref, b1_ref, w2_ref, b2_ref, w_ref, bias_ref,
          task_ref, out_ref):
    f32 = jnp.float32
    S = s_ref[...]
    T = t_ref[...]

    dot = functools.partial(jax.lax.dot_general,
                            preferred_element_type=jnp.float32)

    scores = jnp.maximum(
        dot(S, T, (((1,), (1,)), ((), ()))), 0.0)

    ka = jnp.int32(_K_LOW + 1)
    kb = jnp.int32(_K_LOW + 2)
    maxf = jnp.int32(_MAX_FINITE_BITS)

    def bs_step(_, carry):
        lo_a, hi_a, lo_b, hi_b = carry
        mid_a = lo_a + (hi_a - lo_a) // 2
        mid_b = lo_b + (hi_b - lo_b) // 2
        ta = jax.lax.bitcast_convert_type(mid_a, f32)
        tb = jax.lax.bitcast_convert_type(mid_b, f32)
        cnt_a = jnp.count_nonzero(scores <= ta)
        cnt_b = jnp.count_nonzero(scores <= tb)
        ge_a = cnt_a >= ka
        ge_b = cnt_b >= kb
        lo_a = jnp.where(ge_a, lo_a, mid_a + 1)
        hi_a = jnp.where(ge_a, mid_a, hi_a)
        lo_b = jnp.where(ge_b, lo_b, mid_b + 1)
        hi_b = jnp.where(ge_b, mid_b, hi_b)
        return lo_a, hi_a, lo_b, hi_b

    lo0 = jnp.int32(0)
    _, vk_bits, _, vk1_bits = jax.lax.fori_loop(
        0, 31, bs_step, (lo0, maxf, lo0, maxf))

    vk = jax.lax.bitcast_convert_type(vk_bits, f32)
    vk1 = jax.lax.bitcast_convert_type(vk1_bits, f32)
    md = vk + (vk1 - vk) * f32(0.5)

    mask = (scores > md).astype(f32)
    ones_t = jnp.ones((_T_NUM, 1), f32)
    ones_s = jnp.ones((_S_NUM, 1), f32)
    deg_s = dot(mask, ones_t, (((1,), (0,)), ((), ()))) + 1.0
    deg_t = dot(mask, ones_s, (((0,), (0,)), ((), ()))) + 1.0
    dinv_s = jax.lax.rsqrt(deg_s)
    dinv_t = jax.lax.rsqrt(deg_t)

    W1 = w1_ref[...]
    b1 = b1_ref[...]
    W2 = w2_ref[...]
    b2 = b2_ref[...]

    def agg(hs, ht):
        ms = dot(mask, dinv_t * ht, (((1,), (0,)), ((), ())))
        mt = dot(mask, dinv_s * hs, (((0,), (0,)), ((), ())))
        out_s = dinv_s * (dinv_s * hs + ms)
        out_t = dinv_t * (dinv_t * ht + mt)
        return out_s, out_t

    hs1 = dot(S, W1, (((1,), (0,)), ((), ())))
    ht1 = dot(T, W1, (((1,), (0,)), ((), ())))
    as1, at1 = agg(hs1, ht1)
    h1s = jnp.maximum(as1 + b1, 0.0)
    h1t = jnp.maximum(at1 + b1, 0.0)

    hs2 = dot(h1s, W2, (((1,), (0,)), ((), ())))
    ht2 = dot(h1t, W2, (((1,), (0,)), ((), ())))
    emb_s, emb_t = agg(hs2, ht2)
    emb_s = emb_s + b2
    emb_t = emb_t + b2

    tgt = jnp.sum(emb_t, axis=0, keepdims=True) * f32(1.0 / _T_NUM)
    wv = (w_ref[...] * tgt.T)
    soutar = dot(emb_s, wv, (((1,), (0,)), ((), ()))) + bias_ref[...]
    out = 0.5 * jax.nn.sigmoid(soutar) + 0.5 * jax.nn.sigmoid(task_ref[...])
    out_ref[...] = out


@jax.jit
def kernel(source_stack, target_stack, W1, b1, W2, b2, w, b, task_vec):
    out = pl.pallas_call(
        _body,
        out_shape=jax.ShapeDtypeStruct((_S_NUM, 1), jnp.float32),
    )(source_stack, target_stack, W1, b1.reshape(1, -1), W2,
      b2.reshape(1, -1), w, b.reshape(1, 1), task_vec)
    return out
